# Initial kernel scaffold; baseline (speedup 1.0000x reference)
import jax
import jax.numpy as jnp
from jax.experimental import pallas as pl


def kernel(X_test, X_train, y_train):
    raise NotImplementedError("write your pallas kernel here")


# speedup vs baseline: 75.3001x; 75.3001x over previous
import functools

import jax
import jax.numpy as jnp
from jax.experimental import pallas as pl
from jax.experimental.pallas import tpu as pltpu

_K = 8
_LMASK = 127
_BLK = 2048
_IMAX = jnp.iinfo(jnp.int32).max


def _knn_body(n_real, xt_ref, q_ref, y_ref, out_ref, dist8, meta8, work, flag):
    pid = pl.program_id(0)
    nblk = pl.num_programs(0)
    blk, dim = xt_ref.shape
    nq = q_ref.shape[1]

    @pl.when(pid == 0)
    def _init():
        dist8[...] = jnp.full((_K, nq), jnp.inf, jnp.float32)
        meta8[...] = jnp.full((_K, nq), _IMAX, jnp.int32)

    t = xt_ref[...]
    q = q_ref[...]
    acc = jnp.zeros((blk, nq), jnp.float32)
    for d in range(dim):
        diff = t[:, d:d + 1] - q[d:d + 1, :]
        acc = acc + diff * diff
    dist = jnp.sqrt(acc)

    gcol = pid * blk + jax.lax.broadcasted_iota(jnp.int32, (blk, 1), 0)
    dist = jnp.where(gcol < n_real, dist, jnp.inf)
    meta_col = gcol * (_LMASK + 1) + y_ref[...]

    work[...] = dist
    flag[0] = 1

    def _insert(cd, cm):
        d_l = dist8[_K - 1:_K, :]
        m_l = meta8[_K - 1:_K, :]
        less = (cd < d_l) | ((cd == d_l) & (cm < m_l))
        dist8[_K - 1:_K, :] = jnp.where(less, cd, d_l)
        meta8[_K - 1:_K, :] = jnp.where(less, cm, m_l)
        for i in range(_K - 1, 0, -1):
            a_d = dist8[i:i + 1, :]
            a_m = meta8[i:i + 1, :]
            b_d = dist8[i - 1:i, :]
            b_m = meta8[i - 1:i, :]
            sw = (a_d < b_d) | ((a_d == b_d) & (a_m < b_m))
            dist8[i:i + 1, :] = jnp.where(sw, b_d, a_d)
            dist8[i - 1:i, :] = jnp.where(sw, a_d, b_d)
            meta8[i:i + 1, :] = jnp.where(sw, b_m, a_m)
            meta8[i - 1:i, :] = jnp.where(sw, a_m, b_m)

    for p in range(_K):
        @pl.when(flag[0] == 1)
        def _pass():
            w = work[...]
            m = jnp.min(w, axis=0, keepdims=True)
            tau = dist8[_K - 1:_K, :]
            go = jnp.any(m < tau)
            flag[0] = go.astype(jnp.int32)

            @pl.when(go)
            def _extract():
                selm = jnp.min(jnp.where(w == m, meta_col, _IMAX),
                               axis=0, keepdims=True)
                work[...] = jnp.where(meta_col == selm, jnp.inf, w)
                _insert(m, selm)

    @pl.when(pid == nblk - 1)
    def _vote():
        labels = meta8[...] & _LMASK
        cnt = jnp.zeros((_K, nq), jnp.int32)
        for j in range(_K):
            cnt = cnt + (labels == labels[j:j + 1, :]).astype(jnp.int32)
        key = (_K - cnt) * (_LMASK + 1) + labels
        best = jnp.min(key, axis=0, keepdims=True)
        out_ref[...] = best & _LMASK


def _specs(dim, nq):
    return dict(
        in_specs=[
            pl.BlockSpec((_BLK, dim), lambda i: (i, 0)),
            pl.BlockSpec((dim, nq), lambda i: (0, 0)),
            pl.BlockSpec((_BLK, 1), lambda i: (i, 0)),
        ],
        out_specs=pl.BlockSpec((1, nq), lambda i: (0, 0)),
        out_shape=jax.ShapeDtypeStruct((1, nq), jnp.int32),
        scratch_shapes=[
            pltpu.VMEM((_K, nq), jnp.float32),
            pltpu.VMEM((_K, nq), jnp.int32),
            pltpu.VMEM((_BLK, nq), jnp.float32),
            pltpu.SMEM((1,), jnp.int32),
        ],
    )


def kernel(X_test, X_train, y_train):
    n, dim = X_train.shape
    nq = X_test.shape[0]
    nblk = -(-n // _BLK)
    npad = nblk * _BLK
    xt = jnp.pad(X_train, ((0, npad - n), (0, 0)))
    y = jnp.pad(y_train.astype(jnp.int32), (0, npad - n)).reshape(npad, 1)
    qt = X_test.T
    out = pl.pallas_call(
        functools.partial(_knn_body, n),
        grid=(nblk,),
        **_specs(dim, nq),
    )(xt, qt, y)
    return out.reshape(nq)


# speedup vs baseline: 75.8935x; 1.0079x over previous
import functools

import jax
import jax.numpy as jnp
from jax import lax
from jax.experimental import pallas as pl
from jax.experimental.pallas import tpu as pltpu
from jax.experimental.pallas import tpu_sc as plsc

_K = 8
_LMASK = 127
_BLK = 2048
_IMAX = jnp.iinfo(jnp.int32).max
_QW = 32
_NW = 32


def _knn_body(n_real, xt_ref, q_ref, out_ref, dist8, idx8, work, flag):
    pid = pl.program_id(0)
    nblk = pl.num_programs(0)
    blk, dim = xt_ref.shape
    nq = q_ref.shape[1]

    @pl.when(pid == 0)
    def _init():
        dist8[...] = jnp.full((_K, nq), jnp.inf, jnp.float32)
        idx8[...] = jnp.full((_K, nq), _IMAX, jnp.int32)

    t = xt_ref[...]
    q = q_ref[...]
    acc = jnp.zeros((blk, nq), jnp.float32)
    for d in range(dim):
        diff = t[:, d:d + 1] - q[d:d + 1, :]
        acc = acc + diff * diff
    dist = jnp.sqrt(acc)

    gcol = pid * blk + lax.broadcasted_iota(jnp.int32, (blk, 1), 0)
    dist = jnp.where(gcol < n_real, dist, jnp.inf)

    work[...] = dist
    flag[0] = 1

    def _insert(cd, cm):
        d_l = dist8[_K - 1:_K, :]
        m_l = idx8[_K - 1:_K, :]
        less = (cd < d_l) | ((cd == d_l) & (cm < m_l))
        dist8[_K - 1:_K, :] = jnp.where(less, cd, d_l)
        idx8[_K - 1:_K, :] = jnp.where(less, cm, m_l)
        for i in range(_K - 1, 0, -1):
            a_d = dist8[i:i + 1, :]
            a_m = idx8[i:i + 1, :]
            b_d = dist8[i - 1:i, :]
            b_m = idx8[i - 1:i, :]
            sw = (a_d < b_d) | ((a_d == b_d) & (a_m < b_m))
            dist8[i:i + 1, :] = jnp.where(sw, b_d, a_d)
            dist8[i - 1:i, :] = jnp.where(sw, a_d, b_d)
            idx8[i:i + 1, :] = jnp.where(sw, b_m, a_m)
            idx8[i - 1:i, :] = jnp.where(sw, a_m, b_m)

    for _ in range(_K):
        @pl.when(flag[0] == 1)
        def _pass():
            w = work[...]
            m = jnp.min(w, axis=0, keepdims=True)
            tau = dist8[_K - 1:_K, :]
            go = jnp.any(m < tau)
            flag[0] = go.astype(jnp.int32)

            @pl.when(go)
            def _extract():
                sel = jnp.min(jnp.where(w == m, gcol, _IMAX),
                              axis=0, keepdims=True)
                work[...] = jnp.where(gcol == sel, jnp.inf, w)
                _insert(m, sel)

    @pl.when(pid == nblk - 1)
    def _out():
        out_ref[...] = idx8[...]


def _topk_call(n, dim, nq, nblk):
    return pl.pallas_call(
        functools.partial(_knn_body, n),
        grid=(nblk,),
        in_specs=[
            pl.BlockSpec((_BLK, dim), lambda i: (i, 0)),
            pl.BlockSpec((dim, nq), lambda i: (0, 0)),
        ],
        out_specs=pl.BlockSpec((_K, nq), lambda i: (0, 0)),
        out_shape=jax.ShapeDtypeStruct((_K, nq), jnp.int32),
        scratch_shapes=[
            pltpu.VMEM((_K, nq), jnp.float32),
            pltpu.VMEM((_K, nq), jnp.int32),
            pltpu.VMEM((_BLK, nq), jnp.float32),
            pltpu.SMEM((1,), jnp.int32),
        ],
    )


def _vote_body(idx_hbm, y_hbm, out_hbm, idx_v, lab_v, pred_v, sem):
    wid = lax.axis_index("s") * 2 + lax.axis_index("c")
    base = wid * _QW
    for j in range(_K):
        pltpu.sync_copy(idx_hbm.at[j, pl.ds(base, _QW)], idx_v.at[j])
    for j in range(_K):
        pltpu.async_copy(y_hbm.at[idx_v.at[j]], lab_v.at[j], sem).wait()
    for h in range(_QW // 16):
        labs = [lab_v[j, pl.ds(h * 16, 16)] for j in range(_K)]
        keys = []
        for j in range(_K):
            ne_sum = jnp.zeros((16,), jnp.int32)
            for j2 in range(_K):
                ne_sum = ne_sum + jnp.minimum(jnp.abs(labs[j] - labs[j2]), 1)
            keys.append(ne_sum * (_LMASK + 1) + labs[j])
        best = functools.reduce(jnp.minimum, keys)
        pred_v[pl.ds(h * 16, 16)] = best & _LMASK
    pltpu.sync_copy(pred_v, out_hbm.at[pl.ds(base, _QW)])


def _vote_call(nq):
    mesh = plsc.VectorSubcoreMesh(core_axis_name="c", subcore_axis_name="s")
    return pl.kernel(
        _vote_body,
        out_type=jax.ShapeDtypeStruct((nq,), jnp.int32),
        mesh=mesh,
        scratch_types=[
            pltpu.VMEM((_K, _QW), jnp.int32),
            pltpu.VMEM((_K, _QW), jnp.int32),
            pltpu.VMEM((_QW,), jnp.int32),
            pltpu.SemaphoreType.DMA,
        ],
    )


def kernel(X_test, X_train, y_train):
    n, dim = X_train.shape
    nq = X_test.shape[0]
    nblk = -(-n // _BLK)
    npad = nblk * _BLK
    xt = jnp.pad(X_train, ((0, npad - n), (0, 0)))
    qt = X_test.T
    idx8 = _topk_call(n, dim, nq, nblk)(xt, qt)
    return _vote_call(nq)(idx8, y_train.astype(jnp.int32))
